# Initial kernel scaffold; baseline (speedup 1.0000x reference)
import jax
import jax.numpy as jnp
from jax.experimental import pallas as pl


def kernel(connectivity, coords, EtaR, ShfR):
    raise NotImplementedError("write your pallas kernel here")


# speedup vs baseline: 4.3344x; 4.3344x over previous
import functools

import jax
import jax.numpy as jnp
from jax import lax
from jax.experimental import pallas as pl
from jax.experimental.pallas import tpu as pltpu
from jax.experimental.pallas import tpu_sc as plsc

RC = 5.2
OUTPUT_SIZE = 16

N_BATCH = 100
N_CONN = 32000
N_ATOMS = 1000

NC, NS, L = 2, 16, 16
NW = NC * NS
E_PER_W = N_CONN // NW
N_GROUPS = (E_PER_W + L - 1) // L

_INV_RC = 1.0 / RC
_PI = 3.14159265358979

_S3 = -1.0 / 6.0
_S5 = 1.0 / 120.0
_S7 = -1.0 / 5040.0
_S9 = 1.0 / 362880.0


def _sc_kernel(conn_hbm, coords_hbm, params_hbm, out_hbm,
               conn_v, coords_v, params_v, out_v):
    wid = lax.axis_index("s") * NC + lax.axis_index("c")
    ebase = wid * E_PER_W

    pltpu.sync_copy(params_hbm, params_v)

    iota = lax.iota(jnp.int32, L)
    neg_eta = params_v[pl.ds(OUTPUT_SIZE * L, L)]
    shells = [params_v[pl.ds(j * L, L)] for j in range(OUTPUT_SIZE)]

    def batch_body(b, carry):
        pltpu.sync_copy(
            conn_hbm.at[pl.ds(b * (N_CONN * 2) + ebase * 2, E_PER_W * 2)],
            conn_v)
        pltpu.sync_copy(
            coords_hbm.at[pl.ds(b * (N_ATOMS * 3), N_ATOMS * 3)], coords_v)

        def group_body(g, carry2):
            base = jnp.minimum(g * L, E_PER_W - L)
            idx2 = (base + iota) * 2
            ia = plsc.load_gather(conn_v, [idx2])
            idn = plsc.load_gather(conn_v, [idx2 + 1])
            ia3 = ia * 3
            id3 = idn * 3
            xa = plsc.load_gather(coords_v, [ia3])
            ya = plsc.load_gather(coords_v, [ia3 + 1])
            za = plsc.load_gather(coords_v, [ia3 + 2])
            xd = plsc.load_gather(coords_v, [id3])
            yd = plsc.load_gather(coords_v, [id3 + 1])
            zd = plsc.load_gather(coords_v, [id3 + 2])
            dx = xa - xd
            dy = ya - yd
            dz = za - zd
            r2 = dx * dx + dy * dy + dz * dz
            r2s = jnp.maximum(r2, 1e-24)
            bits = plsc.bitcast(r2s, jnp.int32)
            y = plsc.bitcast(jnp.int32(0x5F3759DF) - (bits >> 1), jnp.float32)
            h = 0.5 * r2s
            y = y * (1.5 - h * y * y)
            y = y * (1.5 - h * y * y)
            y = y * (1.5 - h * y * y)
            d = r2 * y
            u = jnp.minimum(d * _INV_RC, 1.0)
            t = (u - 0.5) * _PI
            t2 = t * t
            p = 1.0 + t2 * (_S3 + t2 * (_S5 + t2 * (_S7 + t2 * _S9)))
            sin_t = t * p
            fc = jnp.where(d <= RC, 0.5 - 0.5 * sin_t, 0.0)
            fcq = fc * 0.25
            erow16 = (base + iota) * OUTPUT_SIZE
            for j in range(OUTPUT_SIZE):
                tj = d - shells[j]
                ej = jnp.exp(neg_eta * (tj * tj))
                plsc.store_scatter(out_v, [erow16 + j], ej * fcq)
            return carry2

        lax.fori_loop(0, N_GROUPS, group_body, 0, unroll=False)
        pltpu.sync_copy(
            out_v,
            out_hbm.at[pl.ds(b * (N_CONN * OUTPUT_SIZE) + ebase * OUTPUT_SIZE,
                             E_PER_W * OUTPUT_SIZE)])
        return carry

    lax.fori_loop(0, N_BATCH, batch_body, 0, unroll=False)


@jax.jit
def _apev(connectivity, coords, EtaR, ShfR):
    conn2 = connectivity.reshape(N_BATCH * N_CONN * 2).astype(jnp.int32)
    coords2 = coords.reshape(N_BATCH * N_ATOMS * 3)
    shf_splat = jnp.repeat(ShfR.astype(jnp.float32), L)
    eta_splat = jnp.broadcast_to(-EtaR.astype(jnp.float32), (L,))
    params = jnp.concatenate([shf_splat, eta_splat])
    mesh = plsc.VectorSubcoreMesh(core_axis_name="c", subcore_axis_name="s")
    run = pl.kernel(
        _sc_kernel,
        out_type=jax.ShapeDtypeStruct((N_BATCH * N_CONN * OUTPUT_SIZE,),
                                      jnp.float32),
        mesh=mesh,
        compiler_params=pltpu.CompilerParams(needs_layout_passes=False),
        scratch_types=[
            pltpu.VMEM((E_PER_W * 2,), jnp.int32),
            pltpu.VMEM((N_ATOMS * 3,), jnp.float32),
            pltpu.VMEM(((OUTPUT_SIZE + 1) * L,), jnp.float32),
            pltpu.VMEM((E_PER_W * OUTPUT_SIZE,), jnp.float32),
        ],
    )
    return run(conn2, coords2, params).reshape(N_BATCH, N_CONN, OUTPUT_SIZE)


def kernel(connectivity, coords, EtaR, ShfR):
    y = _apev(connectivity, coords, EtaR, ShfR)
    return (connectivity, y)


# speedup vs baseline: 5.0195x; 1.1581x over previous
import functools

import jax
import jax.numpy as jnp
from jax import lax
from jax.experimental import pallas as pl
from jax.experimental.pallas import tpu as pltpu
from jax.experimental.pallas import tpu_sc as plsc

RC = 5.2
OUTPUT_SIZE = 16

N_BATCH = 100
N_CONN = 32000
N_ATOMS = 1000

NC, NS, L = 2, 16, 16
NW = NC * NS
EB = 128
N_BLOCKS = N_CONN // EB
BLOCKS_PER_W = -(-N_BLOCKS // NW)
GROUPS_PER_B = EB // L

_INV_RC = 1.0 / RC
_PI = 3.14159265358979

_S3 = -1.0 / 6.0
_S5 = 1.0 / 120.0
_S7 = -1.0 / 5040.0
_S9 = 1.0 / 362880.0


def _sc_kernel(conn_hbm, coords_hbm, params_hbm, out_hbm,
               conn_v, coords_v, params_v, out_v):
    wid = lax.axis_index("s") * NC + lax.axis_index("c")

    pltpu.sync_copy(params_hbm, params_v)

    iota = lax.iota(jnp.int32, L)
    neg_eta = params_v[pl.ds(OUTPUT_SIZE * L, L)]
    shells = [params_v[pl.ds(j * L, L)] for j in range(OUTPUT_SIZE)]

    def batch_body(b, carry):
        pltpu.sync_copy(
            coords_hbm.at[pl.ds(b * (N_ATOMS * 3), N_ATOMS * 3)], coords_v)

        def block_body(t, carry2):
            et = t * NW + wid

            @pl.when(et < N_BLOCKS)
            def _():
                pltpu.sync_copy(
                    conn_hbm.at[pl.ds((b * N_CONN + et * EB) * 2, EB * 2)],
                    conn_v)
                for g in range(GROUPS_PER_B):
                    idx2 = iota * 2 + (g * L * 2)
                    ia = plsc.load_gather(conn_v, [idx2])
                    idn = plsc.load_gather(conn_v, [idx2 + 1])
                    ia3 = ia * 3
                    id3 = idn * 3
                    xa = plsc.load_gather(coords_v, [ia3])
                    ya = plsc.load_gather(coords_v, [ia3 + 1])
                    za = plsc.load_gather(coords_v, [ia3 + 2])
                    xd = plsc.load_gather(coords_v, [id3])
                    yd = plsc.load_gather(coords_v, [id3 + 1])
                    zd = plsc.load_gather(coords_v, [id3 + 2])
                    dx = xa - xd
                    dy = ya - yd
                    dz = za - zd
                    r2 = dx * dx + dy * dy + dz * dz
                    r2s = jnp.maximum(r2, 1e-24)
                    bits = plsc.bitcast(r2s, jnp.int32)
                    y = plsc.bitcast(jnp.int32(0x5F3759DF) - (bits >> 1),
                                     jnp.float32)
                    h = 0.5 * r2s
                    y = y * (1.5 - h * y * y)
                    y = y * (1.5 - h * y * y)
                    y = y * (1.5 - h * y * y)
                    d = r2 * y
                    u = jnp.minimum(d * _INV_RC, 1.0)
                    tt = (u - 0.5) * _PI
                    t2 = tt * tt
                    p = 1.0 + t2 * (_S3 + t2 * (_S5 + t2 * (_S7 + t2 * _S9)))
                    sin_t = tt * p
                    fc = jnp.where(d <= RC, 0.5 - 0.5 * sin_t, 0.0)
                    fcq = fc * 0.25
                    for j in range(OUTPUT_SIZE):
                        tj = d - shells[j]
                        ej = jnp.exp(neg_eta * (tj * tj))
                        out_v[j, pl.ds(g * L, L)] = ej * fcq
                pltpu.sync_copy(
                    out_v,
                    out_hbm.at[b, pl.ds(0, OUTPUT_SIZE),
                               pl.ds(et * EB, EB)])
            return carry2

        lax.fori_loop(0, BLOCKS_PER_W, block_body, 0, unroll=False)
        return carry

    lax.fori_loop(0, N_BATCH, batch_body, 0, unroll=False)


@jax.jit
def _apev(connectivity, coords, EtaR, ShfR):
    conn2 = connectivity.reshape(N_BATCH * N_CONN * 2).astype(jnp.int32)
    coords2 = coords.reshape(N_BATCH * N_ATOMS * 3)
    shf_splat = jnp.repeat(ShfR.astype(jnp.float32), L)
    eta_splat = jnp.broadcast_to(-EtaR.astype(jnp.float32), (L,))
    params = jnp.concatenate([shf_splat, eta_splat])
    mesh = plsc.VectorSubcoreMesh(core_axis_name="c", subcore_axis_name="s")
    run = pl.kernel(
        _sc_kernel,
        out_type=jax.ShapeDtypeStruct((N_BATCH, OUTPUT_SIZE, N_CONN),
                                      jnp.float32),
        mesh=mesh,
        compiler_params=pltpu.CompilerParams(needs_layout_passes=False),
        scratch_types=[
            pltpu.VMEM((EB * 2,), jnp.int32),
            pltpu.VMEM((N_ATOMS * 3,), jnp.float32),
            pltpu.VMEM(((OUTPUT_SIZE + 1) * L,), jnp.float32),
            pltpu.VMEM((OUTPUT_SIZE, EB), jnp.float32),
        ],
    )
    yt = run(conn2, coords2, params)
    return jnp.swapaxes(yt, 1, 2)


def kernel(connectivity, coords, EtaR, ShfR):
    y = _apev(connectivity, coords, EtaR, ShfR)
    return (connectivity, y)


# speedup vs baseline: 22.7186x; 4.5260x over previous
import functools

import jax
import jax.numpy as jnp
from jax import lax
from jax.experimental import pallas as pl
from jax.experimental.pallas import tpu as pltpu
from jax.experimental.pallas import tpu_sc as plsc

RC = 5.2
OUTPUT_SIZE = 16

N_BATCH = 100
N_CONN = 32000
N_ATOMS = 1000

NC, NS, L = 2, 16, 16
NW = NC * NS
EB = 128
N_BLOCKS = N_CONN // EB
BLOCKS_PER_W = -(-N_BLOCKS // NW)
GROUPS_PER_B = EB // L

_INV_RC = 1.0 / RC
_PI = 3.14159265358979

_S3 = -1.0 / 6.0
_S5 = 1.0 / 120.0
_S7 = -1.0 / 5040.0
_S9 = 1.0 / 362880.0


def _sc_kernel(conn_hbm, coords_hbm, params_hbm, out_hbm,
               conn_v, coords_v, params_v, out_v):
    wid = lax.axis_index("s") * NC + lax.axis_index("c")

    pltpu.sync_copy(params_hbm, params_v)

    iota = lax.iota(jnp.int32, L)
    neg_eta = params_v[pl.ds(OUTPUT_SIZE * L, L)]
    shells = [params_v[pl.ds(j * L, L)] for j in range(OUTPUT_SIZE)]

    def batch_body(b, carry):
        pltpu.sync_copy(
            coords_hbm.at[pl.ds(b * (N_ATOMS * 3), N_ATOMS * 3)], coords_v)

        def block_body(t, carry2):
            et = t * NW + wid

            @pl.when(et < N_BLOCKS)
            def _():
                pltpu.sync_copy(
                    conn_hbm.at[b, pl.ds(0, 2), pl.ds(et * EB, EB)],
                    conn_v)
                for g in range(GROUPS_PER_B):
                    ia = conn_v[0, pl.ds(g * L, L)]
                    idn = conn_v[1, pl.ds(g * L, L)]
                    ia3 = ia * 3
                    id3 = idn * 3
                    xa = plsc.load_gather(coords_v, [ia3])
                    ya = plsc.load_gather(coords_v, [ia3 + 1])
                    za = plsc.load_gather(coords_v, [ia3 + 2])
                    xd = plsc.load_gather(coords_v, [id3])
                    yd = plsc.load_gather(coords_v, [id3 + 1])
                    zd = plsc.load_gather(coords_v, [id3 + 2])
                    dx = xa - xd
                    dy = ya - yd
                    dz = za - zd
                    r2 = dx * dx + dy * dy + dz * dz
                    r2s = jnp.maximum(r2, 1e-24)
                    bits = plsc.bitcast(r2s, jnp.int32)
                    y = plsc.bitcast(jnp.int32(0x5F3759DF) - (bits >> 1),
                                     jnp.float32)
                    h = 0.5 * r2s
                    y = y * (1.5 - h * y * y)
                    y = y * (1.5 - h * y * y)
                    y = y * (1.5 - h * y * y)
                    d = r2 * y
                    u = jnp.minimum(d * _INV_RC, 1.0)
                    tt = (u - 0.5) * _PI
                    t2 = tt * tt
                    p = 1.0 + t2 * (_S3 + t2 * (_S5 + t2 * (_S7 + t2 * _S9)))
                    sin_t = tt * p
                    fc = jnp.where(d <= RC, 0.5 - 0.5 * sin_t, 0.0)
                    fcq = fc * 0.25
                    for j in range(OUTPUT_SIZE):
                        tj = d - shells[j]
                        ej = jnp.exp(neg_eta * (tj * tj))
                        out_v[j, pl.ds(g * L, L)] = ej * fcq
                pltpu.sync_copy(
                    out_v,
                    out_hbm.at[b, pl.ds(0, OUTPUT_SIZE),
                               pl.ds(et * EB, EB)])
            return carry2

        lax.fori_loop(0, BLOCKS_PER_W, block_body, 0, unroll=False)
        return carry

    lax.fori_loop(0, N_BATCH, batch_body, 0, unroll=False)


@jax.jit
def _apev(connectivity, coords, EtaR, ShfR):
    conn2 = jnp.swapaxes(connectivity.astype(jnp.int32), 1, 2)
    coords2 = coords.reshape(N_BATCH * N_ATOMS * 3)
    shf_splat = jnp.repeat(ShfR.astype(jnp.float32), L)
    eta_splat = jnp.broadcast_to(-EtaR.astype(jnp.float32), (L,))
    params = jnp.concatenate([shf_splat, eta_splat])
    mesh = plsc.VectorSubcoreMesh(core_axis_name="c", subcore_axis_name="s")
    run = pl.kernel(
        _sc_kernel,
        out_type=jax.ShapeDtypeStruct((N_BATCH, OUTPUT_SIZE, N_CONN),
                                      jnp.float32),
        mesh=mesh,
        compiler_params=pltpu.CompilerParams(needs_layout_passes=False),
        scratch_types=[
            pltpu.VMEM((2, EB), jnp.int32),
            pltpu.VMEM((N_ATOMS * 3,), jnp.float32),
            pltpu.VMEM(((OUTPUT_SIZE + 1) * L,), jnp.float32),
            pltpu.VMEM((OUTPUT_SIZE, EB), jnp.float32),
        ],
    )
    yt = run(conn2, coords2, params)
    return jnp.swapaxes(yt, 1, 2)


def kernel(connectivity, coords, EtaR, ShfR):
    y = _apev(connectivity, coords, EtaR, ShfR)
    return (connectivity, y)
